# Initial kernel scaffold; baseline (speedup 1.0000x reference)
import jax
import jax.numpy as jnp
from jax.experimental import pallas as pl


def kernel(x, edge_index, edge_weight, edge_attr, mlp_W1, mlp_b1, mlp_W2, mlp_b2, lin1_W, lin2_W, lin2_b, lin_W, lin_b):
    raise NotImplementedError("write your pallas kernel here")


# speedup vs baseline: 1.4283x; 1.4283x over previous
import functools
from math import pi as PI

import numpy as np
import jax
import jax.numpy as jnp
from jax import lax
from jax.experimental import pallas as pl
from jax.experimental.pallas import tpu as pltpu
from jax.experimental.pallas import tpu_sc as plsc

N_NODES = 10000
N_EDGES = 320000
HIDDEN = 128
NG = 50
NF = 128
CUTOFF = 10.0
SHIFT = float(np.log(2.0))

CHUNK = 128
N_CHUNKS = N_EDGES // CHUNK
N_WORKERS = 32
CHUNKS_PER_WORKER = -(-N_CHUNKS // N_WORKERS)
N_ACC = 10240
ROWS_PER_TILE = N_ACC // 16
ZROWS = 16


def _ssp(v):
    return jnp.maximum(v, 0.0) + jnp.log1p(jnp.exp(-jnp.abs(v))) - SHIFT


def _filt_body(ea, ew, w1, b1, w2, b2, o):
    h = lax.dot_general(ea[...], w1[...], (((1,), (1,)), ((), ())),
                        preferred_element_type=jnp.float32)
    h = _ssp(h + b1[...])
    h = lax.dot_general(h, w2[...], (((1,), (1,)), ((), ())),
                        preferred_element_type=jnp.float32)
    h = h + b2[...]
    c = 0.5 * (jnp.cos(ew[...] * (PI / CUTOFF)) + 1.0)
    o[...] = h * c


def _tc_filter(edge_attr, ew2, w1, b1, w2, b2):
    be = 2000
    return pl.pallas_call(
        _filt_body,
        grid=(N_EDGES // be,),
        in_specs=[
            pl.BlockSpec((be, NG), lambda i: (i, 0)),
            pl.BlockSpec((be, 1), lambda i: (i, 0)),
            pl.BlockSpec((NF, NG), lambda i: (0, 0)),
            pl.BlockSpec((1, NF), lambda i: (0, 0)),
            pl.BlockSpec((NF, NF), lambda i: (0, 0)),
            pl.BlockSpec((1, NF), lambda i: (0, 0)),
        ],
        out_specs=pl.BlockSpec((be, NF), lambda i: (i, 0)),
        out_shape=jax.ShapeDtypeStruct((N_EDGES, NF), jnp.float32),
    )(edge_attr, ew2, w1, b1, w2, b2)


def _lin1_body(x, w, o):
    o[...] = lax.dot_general(x[...], w[...], (((1,), (1,)), ((), ())),
                             preferred_element_type=jnp.float32)


def _tc_lin1(x, w):
    bn = 2000
    return pl.pallas_call(
        _lin1_body,
        grid=(N_NODES // bn,),
        in_specs=[
            pl.BlockSpec((bn, HIDDEN), lambda i: (i, 0)),
            pl.BlockSpec((NF, HIDDEN), lambda i: (0, 0)),
        ],
        out_specs=pl.BlockSpec((bn, NF), lambda i: (i, 0)),
        out_shape=jax.ShapeDtypeStruct((N_NODES, NF), jnp.float32),
    )(x, w)


def _sc_body(xl_hbm, w_hbm, src_hbm, dst_hbm, out_hbm,
             srcb, dstb, xb, wb, zb, acc, sem):
    cid = lax.axis_index("c")
    sid = lax.axis_index("s")
    wid = sid * 2 + cid

    z16 = jnp.zeros((16,), jnp.float32)

    def zrow(r, carry):
        for k in range(8):
            zb[r, pl.ds(k * 16, 16)] = z16
        return carry

    lax.fori_loop(0, ZROWS, zrow, 0)
    for j in range(ROWS_PER_TILE // ZROWS):
        pltpu.sync_copy(zb, acc.at[pl.ds(sid * ROWS_PER_TILE + j * ZROWS, ZROWS)])
    plsc.subcore_barrier()

    def body(i, carry):
        c = i * N_WORKERS + wid

        @pl.when(c < N_CHUNKS)
        def _():
            base = c * CHUNK
            pltpu.sync_copy(src_hbm.at[pl.ds(base, CHUNK)], srcb)
            pltpu.sync_copy(dst_hbm.at[pl.ds(base, CHUNK)], dstb)
            pltpu.async_copy(xl_hbm.at[srcb], xb, sem).wait()
            pltpu.sync_copy(w_hbm.at[pl.ds(base, CHUNK)], wb)

            def mrow(r, carry2):
                for k in range(8):
                    s = pl.ds(k * 16, 16)
                    wb[r, s] = wb[r, s] * xb[r, s]
                return carry2

            lax.fori_loop(0, CHUNK, mrow, 0)
            pltpu.sync_copy(wb, acc.at[dstb], add=True)

        return carry

    lax.fori_loop(0, CHUNKS_PER_WORKER, body, 0)
    plsc.subcore_barrier()
    pltpu.sync_copy(acc.at[pl.ds(sid * ROWS_PER_TILE, ROWS_PER_TILE)],
                    out_hbm.at[cid, pl.ds(sid * ROWS_PER_TILE, ROWS_PER_TILE)])


def _sc_msg_agg(xl, w, src, dst):
    mesh = plsc.VectorSubcoreMesh(core_axis_name="c", subcore_axis_name="s")
    f = functools.partial(
        pl.kernel,
        mesh=mesh,
        out_type=jax.ShapeDtypeStruct((2, N_ACC, NF), jnp.float32),
        scratch_types=[
            pltpu.VMEM((CHUNK,), jnp.int32),
            pltpu.VMEM((CHUNK,), jnp.int32),
            pltpu.VMEM((CHUNK, NF), jnp.float32),
            pltpu.VMEM((CHUNK, NF), jnp.float32),
            pltpu.VMEM((ZROWS, NF), jnp.float32),
            pltpu.VMEM_SHARED((N_ACC, NF), jnp.float32),
            pltpu.SemaphoreType.DMA,
        ],
    )(_sc_body)
    return f(xl, w, src, dst)


def _tail_body(p, w2, b2, w, b, o):
    q = p[...]
    s = q[0] + q[1]
    h = lax.dot_general(s, w2[...], (((1,), (1,)), ((), ())),
                        preferred_element_type=jnp.float32)
    h = _ssp(h + b2[...])
    h = lax.dot_general(h, w[...], (((1,), (1,)), ((), ())),
                        preferred_element_type=jnp.float32)
    o[...] = h + b[...]


def _tc_tail(p, w2, b2, w, b):
    bn = 2000
    return pl.pallas_call(
        _tail_body,
        grid=(N_NODES // bn,),
        in_specs=[
            pl.BlockSpec((2, bn, NF), lambda i: (0, i, 0)),
            pl.BlockSpec((HIDDEN, NF), lambda i: (0, 0)),
            pl.BlockSpec((1, HIDDEN), lambda i: (0, 0)),
            pl.BlockSpec((HIDDEN, HIDDEN), lambda i: (0, 0)),
            pl.BlockSpec((1, HIDDEN), lambda i: (0, 0)),
        ],
        out_specs=pl.BlockSpec((bn, HIDDEN), lambda i: (i, 0)),
        out_shape=jax.ShapeDtypeStruct((N_NODES, HIDDEN), jnp.float32),
    )(p, w2, b2, w, b)


def kernel(x, edge_index, edge_weight, edge_attr,
           mlp_W1, mlp_b1, mlp_W2, mlp_b2,
           lin1_W, lin2_W, lin2_b, lin_W, lin_b):
    src = edge_index[0].astype(jnp.int32)
    dst = edge_index[1].astype(jnp.int32)
    ew2 = edge_weight.reshape(N_EDGES, 1)
    w = _tc_filter(edge_attr, ew2, mlp_W1, mlp_b1.reshape(1, NF),
                   mlp_W2, mlp_b2.reshape(1, NF))
    xl = _tc_lin1(x, lin1_W)
    partials = _sc_msg_agg(xl, w, src, dst)
    return _tc_tail(partials, lin2_W, lin2_b.reshape(1, NF),
                    lin_W, lin_b.reshape(1, HIDDEN))


# speedup vs baseline: 2.1409x; 1.4990x over previous
import functools
from math import pi as PI

import numpy as np
import jax
import jax.numpy as jnp
from jax import lax
from jax.experimental import pallas as pl
from jax.experimental.pallas import tpu as pltpu
from jax.experimental.pallas import tpu_sc as plsc

N_NODES = 10000
N_EDGES = 320000
HIDDEN = 128
NG = 50
NF = 128
CUTOFF = 10.0
SHIFT = float(np.log(2.0))

CHUNK = 128
N_CHUNKS = N_EDGES // CHUNK
N_WORKERS = 32
CHUNKS_PER_WORKER = -(-N_CHUNKS // N_WORKERS)
N_ACC = 10240
ROWS_PER_TILE = N_ACC // 16
ZROWS = 16


def _ssp(v):
    return jnp.maximum(v, 0.0) + jnp.log1p(jnp.exp(-jnp.abs(v))) - SHIFT


def _filt_body(ea, cenv, w1, b1, w2, b2, o):
    h = lax.dot_general(ea[...], w1[...], (((1,), (1,)), ((), ())),
                        preferred_element_type=jnp.float32)
    h = _ssp(h + b1[...])
    h = lax.dot_general(h, w2[...], (((1,), (1,)), ((), ())),
                        preferred_element_type=jnp.float32)
    h = h + b2[...]
    o[...] = h * cenv[...]


def _cos_body(ew, o):
    o[...] = 0.5 * (jnp.cos(ew[...] * (PI / CUTOFF)) + 1.0)


def _tc_cos(ew):
    rows = N_EDGES // 128
    return pl.pallas_call(
        _cos_body,
        out_shape=jax.ShapeDtypeStruct((rows, 128), jnp.float32),
    )(ew.reshape(rows, 128))


def _tc_filter(edge_attr, ew2, w1, b1, w2, b2):
    be = 2000
    return pl.pallas_call(
        _filt_body,
        grid=(N_EDGES // be,),
        in_specs=[
            pl.BlockSpec((be, NG), lambda i: (i, 0)),
            pl.BlockSpec((be, 1), lambda i: (i, 0)),
            pl.BlockSpec((NF, NG), lambda i: (0, 0)),
            pl.BlockSpec((1, NF), lambda i: (0, 0)),
            pl.BlockSpec((NF, NF), lambda i: (0, 0)),
            pl.BlockSpec((1, NF), lambda i: (0, 0)),
        ],
        out_specs=pl.BlockSpec((be, NF), lambda i: (i, 0)),
        out_shape=jax.ShapeDtypeStruct((N_EDGES, NF), jnp.float32),
    )(edge_attr, ew2, w1, b1, w2, b2)


def _lin1_body(x, w, o):
    o[...] = lax.dot_general(x[...], w[...], (((1,), (1,)), ((), ())),
                             preferred_element_type=jnp.float32)


def _tc_lin1(x, w):
    bn = 2000
    return pl.pallas_call(
        _lin1_body,
        grid=(N_NODES // bn,),
        in_specs=[
            pl.BlockSpec((bn, HIDDEN), lambda i: (i, 0)),
            pl.BlockSpec((NF, HIDDEN), lambda i: (0, 0)),
        ],
        out_specs=pl.BlockSpec((bn, NF), lambda i: (i, 0)),
        out_shape=jax.ShapeDtypeStruct((N_NODES, NF), jnp.float32),
    )(x, w)


def _sc_body(xl_hbm, w_hbm, src_hbm, dst_hbm, out_hbm,
             srcb, dstb, xb, wb, zb, acc, sem):
    cid = lax.axis_index("c")
    sid = lax.axis_index("s")
    wid = sid * 2 + cid

    z16 = jnp.zeros((16,), jnp.float32)

    def zrow(r, carry):
        for k in range(8):
            zb[r, pl.ds(k * 16, 16)] = z16
        return carry

    lax.fori_loop(0, ZROWS, zrow, 0)
    for j in range(ROWS_PER_TILE // ZROWS):
        pltpu.sync_copy(zb, acc.at[pl.ds(sid * ROWS_PER_TILE + j * ZROWS, ZROWS)])
    plsc.subcore_barrier()

    def body(i, carry):
        c = i * N_WORKERS + wid

        @pl.when(c < N_CHUNKS)
        def _():
            base = c * CHUNK
            pltpu.sync_copy(src_hbm.at[pl.ds(base, CHUNK)], srcb)
            pltpu.sync_copy(dst_hbm.at[pl.ds(base, CHUNK)], dstb)
            pltpu.async_copy(xl_hbm.at[srcb], xb, sem).wait()
            pltpu.sync_copy(w_hbm.at[pl.ds(base, CHUNK)], wb)

            def mrow(r, carry2):
                for k in range(8):
                    s = pl.ds(k * 16, 16)
                    wb[r, s] = wb[r, s] * xb[r, s]
                return carry2

            lax.fori_loop(0, CHUNK, mrow, 0)
            pltpu.sync_copy(wb, acc.at[dstb], add=True)

        return carry

    lax.fori_loop(0, CHUNKS_PER_WORKER, body, 0)
    plsc.subcore_barrier()
    pltpu.sync_copy(acc.at[pl.ds(sid * ROWS_PER_TILE, ROWS_PER_TILE)],
                    out_hbm.at[cid, pl.ds(sid * ROWS_PER_TILE, ROWS_PER_TILE)])


def _sc_msg_agg(xl, w, src, dst):
    mesh = plsc.VectorSubcoreMesh(core_axis_name="c", subcore_axis_name="s")
    f = functools.partial(
        pl.kernel,
        mesh=mesh,
        out_type=jax.ShapeDtypeStruct((2, N_ACC, NF), jnp.float32),
        scratch_types=[
            pltpu.VMEM((CHUNK,), jnp.int32),
            pltpu.VMEM((CHUNK,), jnp.int32),
            pltpu.VMEM((CHUNK, NF), jnp.float32),
            pltpu.VMEM((CHUNK, NF), jnp.float32),
            pltpu.VMEM((ZROWS, NF), jnp.float32),
            pltpu.VMEM_SHARED((N_ACC, NF), jnp.float32),
            pltpu.SemaphoreType.DMA,
        ],
    )(_sc_body)
    return f(xl, w, src, dst)


def _tail_body(p, w2, b2, w, b, o):
    q = p[...]
    s = q[0] + q[1]
    h = lax.dot_general(s, w2[...], (((1,), (1,)), ((), ())),
                        preferred_element_type=jnp.float32)
    h = _ssp(h + b2[...])
    h = lax.dot_general(h, w[...], (((1,), (1,)), ((), ())),
                        preferred_element_type=jnp.float32)
    o[...] = h + b[...]


def _tc_tail(p, w2, b2, w, b):
    bn = 2000
    return pl.pallas_call(
        _tail_body,
        grid=(N_NODES // bn,),
        in_specs=[
            pl.BlockSpec((2, bn, NF), lambda i: (0, i, 0)),
            pl.BlockSpec((HIDDEN, NF), lambda i: (0, 0)),
            pl.BlockSpec((1, HIDDEN), lambda i: (0, 0)),
            pl.BlockSpec((HIDDEN, HIDDEN), lambda i: (0, 0)),
            pl.BlockSpec((1, HIDDEN), lambda i: (0, 0)),
        ],
        out_specs=pl.BlockSpec((bn, HIDDEN), lambda i: (i, 0)),
        out_shape=jax.ShapeDtypeStruct((N_NODES, HIDDEN), jnp.float32),
    )(p, w2, b2, w, b)


def kernel(x, edge_index, edge_weight, edge_attr,
           mlp_W1, mlp_b1, mlp_W2, mlp_b2,
           lin1_W, lin2_W, lin2_b, lin_W, lin_b):
    src = edge_index[0].astype(jnp.int32)
    dst = edge_index[1].astype(jnp.int32)
    ew2 = _tc_cos(edge_weight).reshape(N_EDGES, 1)
    w = _tc_filter(edge_attr, ew2, mlp_W1, mlp_b1.reshape(1, NF),
                   mlp_W2, mlp_b2.reshape(1, NF))
    xl = _tc_lin1(x, lin1_W)
    partials = _sc_msg_agg(xl, w, src, dst)
    return _tc_tail(partials, lin2_W, lin2_b.reshape(1, NF),
                    lin_W, lin_b.reshape(1, HIDDEN))


# speedup vs baseline: 2.6115x; 1.2198x over previous
import functools
from math import pi as PI

import numpy as np
import jax
import jax.numpy as jnp
from jax import lax
from jax.experimental import pallas as pl
from jax.experimental.pallas import tpu as pltpu
from jax.experimental.pallas import tpu_sc as plsc

N_NODES = 10000
N_EDGES = 320000
HIDDEN = 128
NG = 50
NF = 128
CUTOFF = 10.0
SHIFT = float(np.log(2.0))

CHUNK = 128
N_CHUNKS = N_EDGES // CHUNK
N_WORKERS = 32
CHUNKS_PER_WORKER = -(-N_CHUNKS // N_WORKERS)
N_ACC = 10240
ROWS_PER_TILE = N_ACC // 16
ZROWS = 16


def _ssp(v):
    return jnp.maximum(v, 0.0) + jnp.log1p(jnp.exp(-jnp.abs(v))) - SHIFT


def _filt_body(ea, cenv, w1, b1, w2, b2, o):
    h = lax.dot_general(ea[...], w1[...], (((1,), (1,)), ((), ())),
                        preferred_element_type=jnp.float32)
    h = _ssp(h + b1[...])
    h = lax.dot_general(h, w2[...], (((1,), (1,)), ((), ())),
                        preferred_element_type=jnp.float32)
    h = h + b2[...]
    ct = jnp.transpose(cenv[...])
    for g in range(ct.shape[1]):
        sl = pl.ds(g * 128, 128)
        o[sl, :] = h[g * 128:(g + 1) * 128, :] * ct[:, g:g + 1]


def _cos_body(ew, o):
    o[...] = 0.5 * (jnp.cos(ew[...] * (PI / CUTOFF)) + 1.0)


def _tc_cos(ew):
    rows = N_EDGES // 128
    return pl.pallas_call(
        _cos_body,
        out_shape=jax.ShapeDtypeStruct((rows, 128), jnp.float32),
    )(ew.reshape(rows, 128))


def _tc_filter(edge_attr, cenv, w1, b1, w2, b2):
    be = 5120
    steps = -(-N_EDGES // be)
    return pl.pallas_call(
        _filt_body,
        grid=(steps,),
        in_specs=[
            pl.BlockSpec((be, NG), lambda i: (i, 0)),
            pl.BlockSpec((be // 128, 128), lambda i: (i, 0)),
            pl.BlockSpec((NF, NG), lambda i: (0, 0)),
            pl.BlockSpec((1, NF), lambda i: (0, 0)),
            pl.BlockSpec((NF, NF), lambda i: (0, 0)),
            pl.BlockSpec((1, NF), lambda i: (0, 0)),
        ],
        out_specs=pl.BlockSpec((be, NF), lambda i: (i, 0)),
        out_shape=jax.ShapeDtypeStruct((N_EDGES, NF), jnp.float32),
    )(edge_attr, cenv, w1, b1, w2, b2)


def _lin1_body(x, w, o):
    o[...] = lax.dot_general(x[...], w[...], (((1,), (1,)), ((), ())),
                             preferred_element_type=jnp.float32)


def _tc_lin1(x, w):
    bn = 2000
    return pl.pallas_call(
        _lin1_body,
        grid=(N_NODES // bn,),
        in_specs=[
            pl.BlockSpec((bn, HIDDEN), lambda i: (i, 0)),
            pl.BlockSpec((NF, HIDDEN), lambda i: (0, 0)),
        ],
        out_specs=pl.BlockSpec((bn, NF), lambda i: (i, 0)),
        out_shape=jax.ShapeDtypeStruct((N_NODES, NF), jnp.float32),
    )(x, w)


def _sc_body(xl_hbm, w_hbm, src_hbm, dst_hbm, out_hbm,
             srcb, dstb, xb, wb, zb, acc, sem):
    cid = lax.axis_index("c")
    sid = lax.axis_index("s")
    wid = sid * 2 + cid

    z16 = jnp.zeros((16,), jnp.float32)

    def zrow(r, carry):
        for k in range(8):
            zb[r, pl.ds(k * 16, 16)] = z16
        return carry

    lax.fori_loop(0, ZROWS, zrow, 0)
    for j in range(ROWS_PER_TILE // ZROWS):
        pltpu.sync_copy(zb, acc.at[pl.ds(sid * ROWS_PER_TILE + j * ZROWS, ZROWS)])
    plsc.subcore_barrier()

    def body(i, carry):
        c = i * N_WORKERS + wid

        @pl.when(c < N_CHUNKS)
        def _():
            base = c * CHUNK
            pltpu.sync_copy(src_hbm.at[pl.ds(base, CHUNK)], srcb)
            pltpu.sync_copy(dst_hbm.at[pl.ds(base, CHUNK)], dstb)
            pltpu.async_copy(xl_hbm.at[srcb], xb, sem).wait()
            pltpu.sync_copy(w_hbm.at[pl.ds(base, CHUNK)], wb)

            def mrow(r, carry2):
                for k in range(8):
                    s = pl.ds(k * 16, 16)
                    wb[r, s] = wb[r, s] * xb[r, s]
                return carry2

            lax.fori_loop(0, CHUNK, mrow, 0)
            pltpu.sync_copy(wb, acc.at[dstb], add=True)

        return carry

    lax.fori_loop(0, CHUNKS_PER_WORKER, body, 0)
    plsc.subcore_barrier()
    pltpu.sync_copy(acc.at[pl.ds(sid * ROWS_PER_TILE, ROWS_PER_TILE)],
                    out_hbm.at[cid, pl.ds(sid * ROWS_PER_TILE, ROWS_PER_TILE)])


def _sc_msg_agg(xl, w, src, dst):
    mesh = plsc.VectorSubcoreMesh(core_axis_name="c", subcore_axis_name="s")
    f = functools.partial(
        pl.kernel,
        mesh=mesh,
        out_type=jax.ShapeDtypeStruct((2, N_ACC, NF), jnp.float32),
        scratch_types=[
            pltpu.VMEM((CHUNK,), jnp.int32),
            pltpu.VMEM((CHUNK,), jnp.int32),
            pltpu.VMEM((CHUNK, NF), jnp.float32),
            pltpu.VMEM((CHUNK, NF), jnp.float32),
            pltpu.VMEM((ZROWS, NF), jnp.float32),
            pltpu.VMEM_SHARED((N_ACC, NF), jnp.float32),
            pltpu.SemaphoreType.DMA,
        ],
    )(_sc_body)
    return f(xl, w, src, dst)


def _tail_body(p, w2, b2, w, b, o):
    q = p[...]
    s = q[0] + q[1]
    h = lax.dot_general(s, w2[...], (((1,), (1,)), ((), ())),
                        preferred_element_type=jnp.float32)
    h = _ssp(h + b2[...])
    h = lax.dot_general(h, w[...], (((1,), (1,)), ((), ())),
                        preferred_element_type=jnp.float32)
    o[...] = h + b[...]


def _tc_tail(p, w2, b2, w, b):
    bn = 2000
    return pl.pallas_call(
        _tail_body,
        grid=(N_NODES // bn,),
        in_specs=[
            pl.BlockSpec((2, bn, NF), lambda i: (0, i, 0)),
            pl.BlockSpec((HIDDEN, NF), lambda i: (0, 0)),
            pl.BlockSpec((1, HIDDEN), lambda i: (0, 0)),
            pl.BlockSpec((HIDDEN, HIDDEN), lambda i: (0, 0)),
            pl.BlockSpec((1, HIDDEN), lambda i: (0, 0)),
        ],
        out_specs=pl.BlockSpec((bn, HIDDEN), lambda i: (i, 0)),
        out_shape=jax.ShapeDtypeStruct((N_NODES, HIDDEN), jnp.float32),
    )(p, w2, b2, w, b)


def kernel(x, edge_index, edge_weight, edge_attr,
           mlp_W1, mlp_b1, mlp_W2, mlp_b2,
           lin1_W, lin2_W, lin2_b, lin_W, lin_b):
    src = edge_index[0].astype(jnp.int32)
    dst = edge_index[1].astype(jnp.int32)
    ew2 = _tc_cos(edge_weight)
    w = _tc_filter(edge_attr, ew2, mlp_W1, mlp_b1.reshape(1, NF),
                   mlp_W2, mlp_b2.reshape(1, NF))
    xl = _tc_lin1(x, lin1_W)
    partials = _sc_msg_agg(xl, w, src, dst)
    return _tc_tail(partials, lin2_W, lin2_b.reshape(1, NF),
                    lin_W, lin_b.reshape(1, HIDDEN))
